# baseline (speedup 1.0000x reference)
import functools

import jax
import jax.numpy as jnp
from jax import lax
from jax.experimental import pallas as pl
from jax.experimental.pallas import tpu as pltpu
from jax.experimental.pallas import tpu_sc as plsc

_INT_MIN = -2147483648
_BLOCK_ROWS = 4
_BLOCK_COLS = 4
_MASK_SHAPE = (1024, 1024)
_N = _MASK_SHAPE[0] * _MASK_SHAPE[1]

_NC = 2
_NS = 16
_NW = _NC * _NS
_CHUNK = _N // _NW
_NBIN = 8192
_VECS_PER_CHUNK = _CHUNK // 16
_BINVECS = _NBIN // 16
_SLICE = _NBIN // _NS


def _monotone_key(x):
    b = lax.bitcast_convert_type(x, jnp.int32)
    return jnp.where(b >= 0, b, jnp.int32(_INT_MIN) - b)


def _key_vec(x):
    b = lax.bitcast_convert_type(x, jnp.int32)
    return jnp.where(b >= 0, b, jnp.int32(_INT_MIN) - b)


def _merge_and_write(cid, sid, hist_v, red_v, out_v, shared_h, hist_out):
    pltpu.sync_copy(hist_v, shared_h.at[pl.ds(sid * _NBIN, _NBIN)])
    plsc.subcore_barrier()
    for r in range(_NS):
        pltpu.sync_copy(
            shared_h.at[pl.ds(r * _NBIN + sid * _SLICE, _SLICE)],
            red_v.at[pl.ds(r * _SLICE, _SLICE)],
        )
    for i in range(_SLICE // 16):
        acc = red_v[pl.ds(i * 16, 16)]
        for r in range(1, _NS):
            acc = acc + red_v[pl.ds(r * _SLICE + i * 16, 16)]
        out_v[pl.ds(i * 16, 16)] = acc
    pltpu.sync_copy(
        out_v, hist_out.at[pl.ds(cid * _NBIN + sid * _SLICE, _SLICE)]
    )


def _sc_hist1_body(scores_hbm, hist_out, chunk_v, hist_v, red_v, out_v, shared_h):
    cid = lax.axis_index("c")
    sid = lax.axis_index("s")
    wid = sid * _NC + cid
    pltpu.sync_copy(scores_hbm.at[pl.ds(wid * _CHUNK, _CHUNK)], chunk_v)

    zeros = jnp.zeros((16,), jnp.int32)
    def zbody(i, _):
        hist_v[pl.ds(i * 16, 16)] = zeros
        return 0
    lax.fori_loop(0, _BINVECS, zbody, 0)

    ones = jnp.ones((16,), jnp.int32)
    four096 = jnp.full((16,), 4096, jnp.int32)
    def hbody(i, _):
        key = _key_vec(chunk_v[pl.ds(i * 16, 16)])
        bins = lax.shift_right_logical(key, 19) ^ four096
        plsc.addupdate_scatter(hist_v, [bins], ones)
        return 0
    lax.fori_loop(0, _VECS_PER_CHUNK, hbody, 0)

    _merge_and_write(cid, sid, hist_v, red_v, out_v, shared_h, hist_out)


def _sc_hist2_body(scores_hbm, b1_hbm, hist_out, chunk_v, hist_v, red_v, out_v,
                   b1_v, shared_h):
    cid = lax.axis_index("c")
    sid = lax.axis_index("s")
    wid = sid * _NC + cid
    pltpu.sync_copy(scores_hbm.at[pl.ds(wid * _CHUNK, _CHUNK)], chunk_v)
    pltpu.sync_copy(b1_hbm, b1_v)

    zeros = jnp.zeros((16,), jnp.int32)
    def zbody(i, _):
        hist_v[pl.ds(i * 16, 16)] = zeros
        return 0
    lax.fori_loop(0, _BINVECS, zbody, 0)

    ones = jnp.ones((16,), jnp.int32)
    mask13 = jnp.full((16,), _NBIN - 1, jnp.int32)
    b1x = b1_v[...] ^ jnp.full((16,), 4096, jnp.int32)
    def hbody(i, _):
        key = _key_vec(chunk_v[pl.ds(i * 16, 16)])
        in_bin = lax.shift_right_logical(key, 19) == b1x
        bins = lax.shift_right_logical(key, 6) & mask13
        plsc.addupdate_scatter(hist_v, [bins], ones, mask=in_bin)
        return 0
    lax.fori_loop(0, _VECS_PER_CHUNK, hbody, 0)

    _merge_and_write(cid, sid, hist_v, red_v, out_v, shared_h, hist_out)


def _sc_hist1(scores_flat):
    mesh = plsc.VectorSubcoreMesh(
        core_axis_name="c", subcore_axis_name="s",
        num_cores=_NC, num_subcores=_NS,
    )
    return pl.kernel(
        _sc_hist1_body,
        out_type=jax.ShapeDtypeStruct((_NC * _NBIN,), jnp.int32),
        mesh=mesh,
        compiler_params=pltpu.CompilerParams(needs_layout_passes=False),
        scratch_types=[
            pltpu.VMEM((_CHUNK,), jnp.float32),
            pltpu.VMEM((_NBIN,), jnp.int32),
            pltpu.VMEM((_NBIN,), jnp.int32),
            pltpu.VMEM((_SLICE,), jnp.int32),
            pltpu.VMEM_SHARED((_NS * _NBIN,), jnp.int32),
        ],
    )(scores_flat)


def _sc_hist2(scores_flat, b1_vec):
    mesh = plsc.VectorSubcoreMesh(
        core_axis_name="c", subcore_axis_name="s",
        num_cores=_NC, num_subcores=_NS,
    )
    return pl.kernel(
        _sc_hist2_body,
        out_type=jax.ShapeDtypeStruct((_NC * _NBIN,), jnp.int32),
        mesh=mesh,
        compiler_params=pltpu.CompilerParams(needs_layout_passes=False),
        scratch_types=[
            pltpu.VMEM((_CHUNK,), jnp.float32),
            pltpu.VMEM((_NBIN,), jnp.int32),
            pltpu.VMEM((_NBIN,), jnp.int32),
            pltpu.VMEM((_SLICE,), jnp.int32),
            pltpu.VMEM((16,), jnp.int32),
            pltpu.VMEM_SHARED((_NS * _NBIN,), jnp.int32),
        ],
    )(scores_flat, b1_vec)


def _suffix_and_pick(h, rank):
    hf = h.astype(jnp.float32)
    lt = (
        lax.broadcasted_iota(jnp.int32, (128, 128), 0)
        <= lax.broadcasted_iota(jnp.int32, (128, 128), 1)
    ).astype(jnp.float32)
    lane_cum = jnp.dot(hf, lt, preferred_element_type=jnp.float32,
                       precision=lax.Precision.HIGHEST)
    rowsum = lane_cum[:, 127:128]
    gt = (
        lax.broadcasted_iota(jnp.int32, (64, 64), 1)
        > lax.broadcasted_iota(jnp.int32, (64, 64), 0)
    ).astype(jnp.float32)
    row_above = jnp.dot(gt, rowsum, preferred_element_type=jnp.float32,
                        precision=lax.Precision.HIGHEST)
    suffix = row_above + (rowsum - lane_cum) + hf
    rank_f = rank.astype(jnp.float32)
    ge = (suffix >= rank_f).astype(jnp.int32)
    b = jnp.sum(ge) - 1
    flat_idx = (
        lax.broadcasted_iota(jnp.int32, h.shape, 0) * 128
        + lax.broadcasted_iota(jnp.int32, h.shape, 1)
    )
    sel = (flat_idx == b).astype(jnp.float32)
    s_at_b = jnp.sum(sel * suffix).astype(jnp.int32)
    h_at_b = jnp.sum(sel * hf).astype(jnp.int32)
    return b, s_at_b - h_at_b, h_at_b


def _scan1_kernel(thr_ref, hist_ref, b1vec_ref, meta_ref):
    h = hist_ref[0] + hist_ref[1]
    j = (thr_ref[0] * jnp.float32(_N)).astype(jnp.int32)
    b1, c_above, _ = _suffix_and_pick(h, j)
    b1vec_ref[...] = jnp.full((16,), b1, jnp.int32)
    meta_ref[0] = b1
    meta_ref[1] = j - c_above
    meta_ref[2] = j


def _scan2_kernel(meta_ref, hist_ref, t_ref):
    h = hist_ref[0] + hist_ref[1]
    b1, j1, j = meta_ref[0], meta_ref[1], meta_ref[2]
    b2, _, _ = _suffix_and_pick(h, j1)
    t_v = (b1 << 19) | (b2 << 6)
    t = t_v ^ jnp.int32(_INT_MIN)
    t_ref[0] = jnp.where(j > 0, t, jnp.int32(2147483647))


def _expand_kernel(t_ref, st_ref, o_ref):
    key = _monotone_key(st_ref[...])
    bin_t = (key >= t_ref[0]).astype(jnp.float32)
    col_rep = jnp.repeat(bin_t, _BLOCK_COLS, axis=0)
    rows = col_rep.T
    o_ref[...] = jnp.repeat(rows, _BLOCK_ROWS, axis=0)


def kernel(weight, mask_scores, input, threshold):
    del weight, input
    thr = jnp.reshape(threshold.astype(jnp.float32), (1,))
    scores_flat = jnp.reshape(mask_scores, (_N,))

    hist1 = _sc_hist1(scores_flat)
    b1vec, meta = pl.pallas_call(
        _scan1_kernel,
        in_specs=[
            pl.BlockSpec(memory_space=pltpu.SMEM),
            pl.BlockSpec(memory_space=pltpu.VMEM),
        ],
        out_specs=[
            pl.BlockSpec(memory_space=pltpu.VMEM),
            pl.BlockSpec(memory_space=pltpu.SMEM),
        ],
        out_shape=[
            jax.ShapeDtypeStruct((16,), jnp.int32),
            jax.ShapeDtypeStruct((4,), jnp.int32),
        ],
    )(thr, jnp.reshape(hist1, (_NC, 64, 128)))

    hist2 = _sc_hist2(scores_flat, b1vec)
    t = pl.pallas_call(
        _scan2_kernel,
        in_specs=[
            pl.BlockSpec(memory_space=pltpu.SMEM),
            pl.BlockSpec(memory_space=pltpu.VMEM),
        ],
        out_specs=pl.BlockSpec(memory_space=pltpu.SMEM),
        out_shape=jax.ShapeDtypeStruct((1,), jnp.int32),
    )(meta, jnp.reshape(hist2, (_NC, 64, 128)))

    rows = 128
    grid = (_MASK_SHAPE[0] // rows,)
    scores_t = mask_scores.T
    out = pl.pallas_call(
        _expand_kernel,
        grid=grid,
        in_specs=[
            pl.BlockSpec(memory_space=pltpu.SMEM),
            pl.BlockSpec((_MASK_SHAPE[1], rows), lambda i: (0, i)),
        ],
        out_specs=pl.BlockSpec(
            (rows * _BLOCK_ROWS, _MASK_SHAPE[1] * _BLOCK_COLS), lambda i: (i, 0)
        ),
        out_shape=jax.ShapeDtypeStruct(
            (_MASK_SHAPE[0] * _BLOCK_ROWS, _MASK_SHAPE[1] * _BLOCK_COLS), jnp.float32
        ),
    )(t, scores_t)
    return out


# speedup vs baseline: 1.4938x; 1.4938x over previous
import functools

import jax
import jax.numpy as jnp
from jax import lax
from jax.experimental import pallas as pl
from jax.experimental.pallas import tpu as pltpu
from jax.experimental.pallas import tpu_sc as plsc

_INT_MIN = -2147483648
_BLOCK_ROWS = 4
_BLOCK_COLS = 4
_MASK_SHAPE = (1024, 1024)
_N = _MASK_SHAPE[0] * _MASK_SHAPE[1]

_NC = 2
_NS = 16
_NW = _NC * _NS
_CHUNK = _N // _NW
_NBIN = 8192
_VECS_PER_CHUNK = _CHUNK // 16
_BINVECS = _NBIN // 16
_SLICE = _NBIN // _NS


def _monotone_key(x):
    b = lax.bitcast_convert_type(x, jnp.int32)
    return jnp.where(b >= 0, b, jnp.int32(_INT_MIN) - b)


def _key_vec(x):
    b = lax.bitcast_convert_type(x, jnp.int32)
    return jnp.where(b >= 0, b, jnp.int32(_INT_MIN) - b)


def _merge_and_write(cid, sid, hist_v, red_v, out_v, shared_h, hist_out):
    pltpu.sync_copy(hist_v, shared_h.at[pl.ds(sid * _NBIN, _NBIN)])
    plsc.subcore_barrier()
    for r in range(_NS):
        pltpu.sync_copy(
            shared_h.at[pl.ds(r * _NBIN + sid * _SLICE, _SLICE)],
            red_v.at[pl.ds(r * _SLICE, _SLICE)],
        )
    for i in range(_SLICE // 16):
        acc = red_v[pl.ds(i * 16, 16)]
        for r in range(1, _NS):
            acc = acc + red_v[pl.ds(r * _SLICE + i * 16, 16)]
        out_v[pl.ds(i * 16, 16)] = acc
    pltpu.sync_copy(
        out_v, hist_out.at[pl.ds(cid * _NBIN + sid * _SLICE, _SLICE)]
    )


def _sc_hist1_body(scores_hbm, hist_out, chunk_v, hist_v, red_v, out_v, shared_h):
    cid = lax.axis_index("c")
    sid = lax.axis_index("s")
    wid = sid * _NC + cid
    pltpu.sync_copy(scores_hbm.at[pl.ds(wid * _CHUNK, _CHUNK)], chunk_v)

    zeros = jnp.zeros((16,), jnp.int32)

    @plsc.parallel_loop(0, _BINVECS, unroll=8)
    def _(i):
        hist_v[pl.ds(i * 16, 16)] = zeros

    ones = jnp.ones((16,), jnp.int32)
    four096 = jnp.full((16,), 4096, jnp.int32)

    @plsc.parallel_loop(0, _VECS_PER_CHUNK, unroll=8)
    def _(i):
        key = _key_vec(chunk_v[pl.ds(i * 16, 16)])
        bins = lax.shift_right_logical(key, 19) ^ four096
        plsc.addupdate_scatter(hist_v, [bins], ones)

    _merge_and_write(cid, sid, hist_v, red_v, out_v, shared_h, hist_out)


def _sc_hist2_body(scores_hbm, b1_hbm, hist_out, chunk_v, hist_v, red_v, out_v,
                   b1_v, shared_h):
    cid = lax.axis_index("c")
    sid = lax.axis_index("s")
    wid = sid * _NC + cid
    pltpu.sync_copy(scores_hbm.at[pl.ds(wid * _CHUNK, _CHUNK)], chunk_v)
    pltpu.sync_copy(b1_hbm, b1_v)

    zeros = jnp.zeros((16,), jnp.int32)

    @plsc.parallel_loop(0, _BINVECS, unroll=8)
    def _(i):
        hist_v[pl.ds(i * 16, 16)] = zeros

    ones = jnp.ones((16,), jnp.int32)
    mask13 = jnp.full((16,), _NBIN - 1, jnp.int32)
    b1x = b1_v[...] ^ jnp.full((16,), 4096, jnp.int32)

    @plsc.parallel_loop(0, _VECS_PER_CHUNK, unroll=8)
    def _(i):
        key = _key_vec(chunk_v[pl.ds(i * 16, 16)])
        in_bin = lax.shift_right_logical(key, 19) == b1x
        bins = lax.shift_right_logical(key, 6) & mask13
        plsc.addupdate_scatter(hist_v, [bins], ones, mask=in_bin)

    _merge_and_write(cid, sid, hist_v, red_v, out_v, shared_h, hist_out)


def _sc_hist1(scores_flat):
    mesh = plsc.VectorSubcoreMesh(
        core_axis_name="c", subcore_axis_name="s",
        num_cores=_NC, num_subcores=_NS,
    )
    return pl.kernel(
        _sc_hist1_body,
        out_type=jax.ShapeDtypeStruct((_NC * _NBIN,), jnp.int32),
        mesh=mesh,
        compiler_params=pltpu.CompilerParams(needs_layout_passes=False),
        scratch_types=[
            pltpu.VMEM((_CHUNK,), jnp.float32),
            pltpu.VMEM((_NBIN,), jnp.int32),
            pltpu.VMEM((_NBIN,), jnp.int32),
            pltpu.VMEM((_SLICE,), jnp.int32),
            pltpu.VMEM_SHARED((_NS * _NBIN,), jnp.int32),
        ],
    )(scores_flat)


def _sc_hist2(scores_flat, b1_vec):
    mesh = plsc.VectorSubcoreMesh(
        core_axis_name="c", subcore_axis_name="s",
        num_cores=_NC, num_subcores=_NS,
    )
    return pl.kernel(
        _sc_hist2_body,
        out_type=jax.ShapeDtypeStruct((_NC * _NBIN,), jnp.int32),
        mesh=mesh,
        compiler_params=pltpu.CompilerParams(needs_layout_passes=False),
        scratch_types=[
            pltpu.VMEM((_CHUNK,), jnp.float32),
            pltpu.VMEM((_NBIN,), jnp.int32),
            pltpu.VMEM((_NBIN,), jnp.int32),
            pltpu.VMEM((_SLICE,), jnp.int32),
            pltpu.VMEM((16,), jnp.int32),
            pltpu.VMEM_SHARED((_NS * _NBIN,), jnp.int32),
        ],
    )(scores_flat, b1_vec)


def _suffix_and_pick(h, rank):
    hf = h.astype(jnp.float32)
    lt = (
        lax.broadcasted_iota(jnp.int32, (128, 128), 0)
        <= lax.broadcasted_iota(jnp.int32, (128, 128), 1)
    ).astype(jnp.float32)
    lane_cum = jnp.dot(hf, lt, preferred_element_type=jnp.float32,
                       precision=lax.Precision.HIGHEST)
    rowsum = lane_cum[:, 127:128]
    gt = (
        lax.broadcasted_iota(jnp.int32, (64, 64), 1)
        > lax.broadcasted_iota(jnp.int32, (64, 64), 0)
    ).astype(jnp.float32)
    row_above = jnp.dot(gt, rowsum, preferred_element_type=jnp.float32,
                        precision=lax.Precision.HIGHEST)
    suffix = row_above + (rowsum - lane_cum) + hf
    rank_f = rank.astype(jnp.float32)
    ge = (suffix >= rank_f).astype(jnp.int32)
    b = jnp.sum(ge) - 1
    flat_idx = (
        lax.broadcasted_iota(jnp.int32, h.shape, 0) * 128
        + lax.broadcasted_iota(jnp.int32, h.shape, 1)
    )
    sel = (flat_idx == b).astype(jnp.float32)
    s_at_b = jnp.sum(sel * suffix).astype(jnp.int32)
    h_at_b = jnp.sum(sel * hf).astype(jnp.int32)
    return b, s_at_b - h_at_b, h_at_b


def _scan1_kernel(thr_ref, hist_ref, b1vec_ref, meta_ref):
    h = hist_ref[0] + hist_ref[1]
    j = (thr_ref[0] * jnp.float32(_N)).astype(jnp.int32)
    b1, c_above, _ = _suffix_and_pick(h, j)
    b1vec_ref[...] = jnp.full((16,), b1, jnp.int32)
    meta_ref[0] = b1
    meta_ref[1] = j - c_above
    meta_ref[2] = j


def _scan2_kernel(meta_ref, hist_ref, t_ref):
    h = hist_ref[0] + hist_ref[1]
    b1, j1, j = meta_ref[0], meta_ref[1], meta_ref[2]
    b2, _, _ = _suffix_and_pick(h, j1)
    t_v = (b1 << 19) | (b2 << 6)
    t = t_v ^ jnp.int32(_INT_MIN)
    t_ref[0] = jnp.where(j > 0, t, jnp.int32(2147483647))


def _expand_kernel(t_ref, st_ref, o_ref):
    key = _monotone_key(st_ref[...])
    bin_t = (key >= t_ref[0]).astype(jnp.float32)
    col_rep = jnp.repeat(bin_t, _BLOCK_COLS, axis=0)
    rows = col_rep.T
    o_ref[...] = jnp.repeat(rows, _BLOCK_ROWS, axis=0)


def kernel(weight, mask_scores, input, threshold):
    del weight, input
    thr = jnp.reshape(threshold.astype(jnp.float32), (1,))
    scores_flat = jnp.reshape(mask_scores, (_N,))

    hist1 = _sc_hist1(scores_flat)
    b1vec, meta = pl.pallas_call(
        _scan1_kernel,
        in_specs=[
            pl.BlockSpec(memory_space=pltpu.SMEM),
            pl.BlockSpec(memory_space=pltpu.VMEM),
        ],
        out_specs=[
            pl.BlockSpec(memory_space=pltpu.VMEM),
            pl.BlockSpec(memory_space=pltpu.SMEM),
        ],
        out_shape=[
            jax.ShapeDtypeStruct((16,), jnp.int32),
            jax.ShapeDtypeStruct((4,), jnp.int32),
        ],
    )(thr, jnp.reshape(hist1, (_NC, 64, 128)))

    hist2 = _sc_hist2(scores_flat, b1vec)
    t = pl.pallas_call(
        _scan2_kernel,
        in_specs=[
            pl.BlockSpec(memory_space=pltpu.SMEM),
            pl.BlockSpec(memory_space=pltpu.VMEM),
        ],
        out_specs=pl.BlockSpec(memory_space=pltpu.SMEM),
        out_shape=jax.ShapeDtypeStruct((1,), jnp.int32),
    )(meta, jnp.reshape(hist2, (_NC, 64, 128)))

    rows = 128
    grid = (_MASK_SHAPE[0] // rows,)
    scores_t = mask_scores.T
    out = pl.pallas_call(
        _expand_kernel,
        grid=grid,
        in_specs=[
            pl.BlockSpec(memory_space=pltpu.SMEM),
            pl.BlockSpec((_MASK_SHAPE[1], rows), lambda i: (0, i)),
        ],
        out_specs=pl.BlockSpec(
            (rows * _BLOCK_ROWS, _MASK_SHAPE[1] * _BLOCK_COLS), lambda i: (i, 0)
        ),
        out_shape=jax.ShapeDtypeStruct(
            (_MASK_SHAPE[0] * _BLOCK_ROWS, _MASK_SHAPE[1] * _BLOCK_COLS), jnp.float32
        ),
    )(t, scores_t)
    return out
